# baseline (speedup 1.0000x reference)
import functools

import jax
import jax.numpy as jnp
from jax import lax
from jax.experimental import pallas as pl
from jax.experimental.pallas import tpu as pltpu

B, N, G = 8, 512, 256
H1, H2 = 512, 512
BLK = 1024
NBLK = (B * N) // BLK


def _dense_body(consts_ref, t_ref, x_ref,
                w1h, b1h, w2h, b2h, w3h,
                w1o, b1o, w2o, b2o, w3o,
                out_ref,
                acch_ref, acco_ref, cnt_ref,
                h2h_ref, h2o_ref, ohh_ref, oho_ref):
    k = pl.program_id(0)

    @pl.when(k == 0)
    def _init():
        acch_ref[...] = jnp.zeros_like(acch_ref)
        acco_ref[...] = jnp.zeros_like(acco_ref)
        cnt_ref[...] = jnp.zeros_like(cnt_ref)

    @pl.when(k > 0)
    def _acc_prev():
        acch_ref[...] += jnp.dot(ohh_ref[...], h2h_ref[...],
                                 preferred_element_type=jnp.float32)
        acco_ref[...] += jnp.dot(oho_ref[...], h2o_ref[...],
                                 preferred_element_type=jnp.float32)

    x = x_ref[...].astype(jnp.bfloat16)

    def mlp(w1, b1, w2, b2):
        p = jnp.dot(x, w1[...], preferred_element_type=jnp.float32)
        h = jnp.tanh(p.astype(jnp.bfloat16) + b1[...])
        p2 = jnp.dot(h, w2[...], preferred_element_type=jnp.float32)
        return jnp.tanh(p2.astype(jnp.bfloat16) + b2[...])

    h2h_ref[...] = mlp(w1h, b1h, w2h, b2h)
    h2o_ref[...] = mlp(w1o, b1o, w2o, b2o)

    t = t_ref[k, 0, :]
    iota8 = lax.broadcasted_iota(jnp.int32, (B, BLK), 0)
    rowid = lax.broadcasted_iota(jnp.int32, (B, BLK), 1) + k * BLK
    in_struct = iota8 == rowid // N
    oh_h = jnp.where(in_struct & (t == 0)[None, :], 1.0, 0.0)
    oh_o = jnp.where(in_struct & (t != 0)[None, :], 1.0, 0.0)
    ohh_ref[...] = oh_h.astype(jnp.bfloat16)
    oho_ref[...] = oh_o.astype(jnp.bfloat16)
    cnt_ref[...] += jnp.sum(oh_h, axis=1, keepdims=True)

    @pl.when(k == pl.num_programs(0) - 1)
    def _fin():
        e_h = acch_ref[...] + jnp.dot(ohh_ref[...], h2h_ref[...],
                                      preferred_element_type=jnp.float32)
        e_o = acco_ref[...] + jnp.dot(oho_ref[...], h2o_ref[...],
                                      preferred_element_type=jnp.float32)
        s_h = jnp.sum(e_h * w3h[...], axis=1, keepdims=True)
        s_o = jnp.sum(e_o * w3o[...], axis=1, keepdims=True)
        n_h = cnt_ref[:, :1]
        const = consts_ref[0] * n_h + consts_ref[1] * (N - n_h)
        out_ref[...] = (s_h + s_o + const) * (1.0 / N)


@functools.partial(jax.jit, static_argnames=())
def kernel(types, Gs, W1_H, b1_H, W2_H, b2_H, W3_H, b3_H, off_H,
           W1_O, b1_O, W2_O, b2_O, W3_O, b3_O, off_O):
    x_flat = Gs.reshape(-1, G)
    types3d = types.reshape(NBLK, 1, BLK)
    consts = jnp.stack([b3_H[0] + off_H, b3_O[0] + off_O])

    def full(a):
        return pl.BlockSpec(a.shape, lambda k: (0,) * a.ndim)

    args = [
        types3d, x_flat,
        W1_H.astype(jnp.bfloat16), b1_H.reshape(1, H1).astype(jnp.bfloat16),
        W2_H.astype(jnp.bfloat16), b2_H.reshape(1, H2).astype(jnp.bfloat16),
        W3_H.reshape(1, H2),
        W1_O.astype(jnp.bfloat16), b1_O.reshape(1, H1).astype(jnp.bfloat16),
        W2_O.astype(jnp.bfloat16), b2_O.reshape(1, H2).astype(jnp.bfloat16),
        W3_O.reshape(1, H2),
    ]
    in_specs = [
        pl.BlockSpec((2,), lambda k: (0,), memory_space=pltpu.SMEM),
        full(types3d),
        pl.BlockSpec((BLK, G), lambda k: (k, 0)),
    ] + [full(a) for a in args[2:]]

    out = pl.pallas_call(
        _dense_body,
        grid=(NBLK,),
        in_specs=in_specs,
        out_specs=pl.BlockSpec((B, 1), lambda k: (0, 0)),
        out_shape=jax.ShapeDtypeStruct((B, 1), jnp.float32),
        scratch_shapes=[pltpu.VMEM((B, H2), jnp.float32),
                        pltpu.VMEM((B, H2), jnp.float32),
                        pltpu.VMEM((B, 128), jnp.float32),
                        pltpu.VMEM((BLK, H2), jnp.bfloat16),
                        pltpu.VMEM((BLK, H2), jnp.bfloat16),
                        pltpu.VMEM((B, BLK), jnp.bfloat16),
                        pltpu.VMEM((B, BLK), jnp.bfloat16)],
        compiler_params=pltpu.CompilerParams(
            dimension_semantics=("arbitrary",)),
    )(consts, *args)
    return out


# speedup vs baseline: 1.1538x; 1.1538x over previous
import functools

import jax
import jax.numpy as jnp
from jax import lax
from jax.experimental import pallas as pl
from jax.experimental.pallas import tpu as pltpu

B, N, G = 8, 512, 256
H1, H2 = 512, 512
BLK = 512
NBLK = (B * N) // BLK


def _dense_body(consts_ref, t_ref, x_ref,
                w1h, b1h, w2h, b2h, w3h,
                w1o, b1o, w2o, b2o, w3o,
                out_ref,
                acch_ref, acco_ref, cnt_ref,
                h2h_ref, h2o_ref, ohh_ref, oho_ref):
    k = pl.program_id(0)

    @pl.when(k == 0)
    def _init():
        acch_ref[...] = jnp.zeros_like(acch_ref)
        acco_ref[...] = jnp.zeros_like(acco_ref)
        cnt_ref[...] = jnp.zeros_like(cnt_ref)

    @pl.when(k > 0)
    def _acc_prev():
        acch_ref[...] += jnp.dot(ohh_ref[...], h2h_ref[...],
                                 preferred_element_type=jnp.float32)
        acco_ref[...] += jnp.dot(oho_ref[...], h2o_ref[...],
                                 preferred_element_type=jnp.float32)

    x = x_ref[...].astype(jnp.bfloat16)

    def mlp(w1, b1, w2, b2):
        p = jnp.dot(x, w1[...], preferred_element_type=jnp.float32)
        h = jnp.tanh(p + b1[...]).astype(jnp.bfloat16)
        p2 = jnp.dot(h, w2[...], preferred_element_type=jnp.float32)
        return jnp.tanh(p2 + b2[...]).astype(jnp.bfloat16)

    h2h_ref[...] = mlp(w1h, b1h, w2h, b2h)
    h2o_ref[...] = mlp(w1o, b1o, w2o, b2o)

    t = t_ref[k, 0, :]
    iota8 = lax.broadcasted_iota(jnp.int32, (B, BLK), 0)
    rowid = lax.broadcasted_iota(jnp.int32, (B, BLK), 1) + k * BLK
    in_struct = iota8 == rowid // N
    oh_h = jnp.where(in_struct & (t == 0)[None, :], 1.0, 0.0)
    oh_o = jnp.where(in_struct & (t != 0)[None, :], 1.0, 0.0)
    ohh_ref[...] = oh_h.astype(jnp.bfloat16)
    oho_ref[...] = oh_o.astype(jnp.bfloat16)
    cnt_ref[...] += jnp.sum(oh_h, axis=1, keepdims=True)

    @pl.when(k == pl.num_programs(0) - 1)
    def _fin():
        e_h = acch_ref[...] + jnp.dot(ohh_ref[...], h2h_ref[...],
                                      preferred_element_type=jnp.float32)
        e_o = acco_ref[...] + jnp.dot(oho_ref[...], h2o_ref[...],
                                      preferred_element_type=jnp.float32)
        s_h = jnp.sum(e_h * w3h[...], axis=1, keepdims=True)
        s_o = jnp.sum(e_o * w3o[...], axis=1, keepdims=True)
        n_h = cnt_ref[:, :1]
        const = consts_ref[0] * n_h + consts_ref[1] * (N - n_h)
        out_ref[...] = (s_h + s_o + const) * (1.0 / N)


@functools.partial(jax.jit, static_argnames=())
def kernel(types, Gs, W1_H, b1_H, W2_H, b2_H, W3_H, b3_H, off_H,
           W1_O, b1_O, W2_O, b2_O, W3_O, b3_O, off_O):
    x_flat = Gs.reshape(-1, G)
    types3d = types.reshape(NBLK, 1, BLK)
    consts = jnp.stack([b3_H[0] + off_H, b3_O[0] + off_O])

    def full(a):
        return pl.BlockSpec(a.shape, lambda k: (0,) * a.ndim)

    args = [
        types3d, x_flat,
        W1_H.astype(jnp.bfloat16), b1_H.reshape(1, H1),
        W2_H.astype(jnp.bfloat16), b2_H.reshape(1, H2),
        W3_H.reshape(1, H2),
        W1_O.astype(jnp.bfloat16), b1_O.reshape(1, H1),
        W2_O.astype(jnp.bfloat16), b2_O.reshape(1, H2),
        W3_O.reshape(1, H2),
    ]
    in_specs = [
        pl.BlockSpec((2,), lambda k: (0,), memory_space=pltpu.SMEM),
        full(types3d),
        pl.BlockSpec((BLK, G), lambda k: (k, 0)),
    ] + [full(a) for a in args[2:]]

    out = pl.pallas_call(
        _dense_body,
        grid=(NBLK,),
        in_specs=in_specs,
        out_specs=pl.BlockSpec((B, 1), lambda k: (0, 0)),
        out_shape=jax.ShapeDtypeStruct((B, 1), jnp.float32),
        scratch_shapes=[pltpu.VMEM((B, H2), jnp.float32),
                        pltpu.VMEM((B, H2), jnp.float32),
                        pltpu.VMEM((B, 128), jnp.float32),
                        pltpu.VMEM((BLK, H2), jnp.bfloat16),
                        pltpu.VMEM((BLK, H2), jnp.bfloat16),
                        pltpu.VMEM((B, BLK), jnp.bfloat16),
                        pltpu.VMEM((B, BLK), jnp.bfloat16)],
        compiler_params=pltpu.CompilerParams(
            dimension_semantics=("arbitrary",)),
    )(consts, *args)
    return out
